# Initial kernel scaffold; baseline (speedup 1.0000x reference)
import jax
import jax.numpy as jnp
from jax.experimental import pallas as pl


def kernel(input_ids, token_embedding_weight, position_embedding_weight):
    raise NotImplementedError("write your pallas kernel here")


# speedup vs baseline: 1.1140x; 1.1140x over previous
import functools

import jax
import jax.numpy as jnp
from jax import lax
from jax.experimental import pallas as pl
from jax.experimental.pallas import tpu as pltpu
from jax.experimental.pallas import tpu_sc as plsc

_D = 768
_SEQ = 77
_BATCH = 1024
_NC = 2
_NS = 16
_NW = _NC * _NS
_SPW = _BATCH // _NW
_LANES = 16
_SEQP = 80


def _sc_embed(input_ids, tok_w, pos_w):
  mesh = plsc.VectorSubcoreMesh(core_axis_name="c", subcore_axis_name="s")

  @functools.partial(
      pl.kernel,
      mesh=mesh,
      out_type=jax.ShapeDtypeStruct((_BATCH, _SEQP, _D), jnp.float32),
      scratch_types=[
          pltpu.VMEM((_SPW, _SEQP), jnp.int32),
          pltpu.VMEM((_SEQ, _D), jnp.float32),
          pltpu.VMEM((_SEQP, _D), jnp.float32),
          pltpu.SemaphoreType.DMA,
      ],
  )
  def k(ids_hbm, tab_hbm, pos_hbm, out_hbm, idx_v, pos_v, buf_v, sem):
    wid = lax.axis_index("s") * _NC + lax.axis_index("c")
    pltpu.sync_copy(ids_hbm.at[pl.ds(wid * _SPW, _SPW)], idx_v)
    pltpu.sync_copy(pos_hbm, pos_v)

    def seq_body(j, carry):
      pltpu.async_copy(tab_hbm.at[idx_v.at[j]], buf_v, sem).wait()

      def row_body(r, c2):
        for c in range(_D // _LANES):
          sl = pl.ds(c * _LANES, _LANES)
          buf_v[r, sl] = buf_v[r, sl] + pos_v[r, sl]
        return c2

      lax.fori_loop(0, _SEQ, row_body, 0)
      pltpu.sync_copy(buf_v, out_hbm.at[wid * _SPW + j])
      return carry

    lax.fori_loop(0, _SPW, seq_body, 0)

  return k(input_ids, tok_w, pos_w)


def kernel(input_ids, token_embedding_weight, position_embedding_weight):
  ids = input_ids.astype(jnp.int32)
  ids = jnp.pad(ids, ((0, 0), (0, _SEQP - _SEQ)))
  out = _sc_embed(ids, token_embedding_weight, position_embedding_weight)
  return out[:, :_SEQ, :]
